# baseline (speedup 1.0000x reference)
import functools

import jax
import jax.numpy as jnp
from jax import lax
from jax.experimental import pallas as pl
from jax.experimental.pallas import tpu as pltpu
from jax.experimental.pallas import tpu_sc as plsc

_NC = 2
_NS = 16
_L = 16
_NW = _NC * _NS
_CHUNKS = 4

_TC_COLS = 256
_TC_BLOCK_ROWS = 1024


def _vq_body(nk, per_w, x_hbm, vals_hbm,
             zc_hbm, q_hbm, qh_hbm, x_v, q_v, vals_v,
             sem_out, *sems_in):
    wid = lax.axis_index("c") * _NS + lax.axis_index("s")
    base = wid * per_w
    ch = per_w // _CHUNKS

    in_copies = []
    for c in range(_CHUNKS):
        in_copies.append(pltpu.async_copy(
            x_hbm.at[pl.ds(base + c * ch, ch)],
            x_v.at[pl.ds(c * ch, ch)],
            sems_in[c]))
    pltpu.sync_copy(vals_hbm, vals_v)

    vmin = jnp.full((_L,), jnp.inf, jnp.float32)
    vmax = jnp.full((_L,), -jnp.inf, jnp.float32)
    for j in range(nk // _L):
        vj = vals_v[pl.ds(j * _L, _L)]
        vmin = jnp.minimum(vmin, vj)
        vmax = jnp.maximum(vmax, vj)
    v0s = jnp.min(vmin)
    v63s = jnp.max(vmax)
    zf = jnp.zeros((_L,), jnp.float32)
    v0 = zf + v0s
    istep = (zf + jnp.float32(nk - 1)) / (zf + (v63s - v0s))

    out_copies = []
    for c in range(_CHUNKS):
        in_copies[c].wait()
        sl_v = pl.ds(c * ch, ch)
        sl_h = pl.ds(base + c * ch, ch)
        out_copies.append(pltpu.async_copy(x_v.at[sl_v], zc_hbm.at[sl_h], sem_out))

        @plsc.parallel_loop(c * (ch // _L), (c + 1) * (ch // _L), unroll=4)
        def body(i):
            s = pl.ds(i * _L, _L)
            xv = x_v[s]
            t = (xv - v0) * istep
            ki = jnp.clip(t.astype(jnp.int32), 0, nk - 2)
            k1 = ki + 1
            vk = plsc.load_gather(vals_v, [ki])
            vk1 = plsc.load_gather(vals_v, [k1])
            m = jnp.abs(xv - vk) <= jnp.abs(xv - vk1)
            q_v[s] = jnp.where(m, vk, vk1)

        out_copies.append(pltpu.async_copy(q_v.at[sl_v], q_hbm.at[sl_h], sem_out))
        out_copies.append(pltpu.async_copy(q_v.at[sl_v], qh_hbm.at[sl_h], sem_out))
    for cp in out_copies:
        cp.wait()


@functools.partial(jax.jit, static_argnums=(0, 1))
def _vq_call(n, nk, x, values):
    per_w = n // _NW
    mesh = plsc.VectorSubcoreMesh(core_axis_name="c", subcore_axis_name="s")
    return pl.kernel(
        functools.partial(_vq_body, nk, per_w),
        out_type=(
            jax.ShapeDtypeStruct((n,), jnp.float32),
            jax.ShapeDtypeStruct((n,), jnp.float32),
            jax.ShapeDtypeStruct((n,), jnp.float32),
        ),
        mesh=mesh,
        compiler_params=pltpu.CompilerParams(needs_layout_passes=False),
        scratch_types=[
            pltpu.VMEM((per_w,), jnp.float32),
            pltpu.VMEM((per_w,), jnp.float32),
            pltpu.VMEM((nk,), jnp.float32),
            pltpu.SemaphoreType.DMA,
        ] + [pltpu.SemaphoreType.DMA] * _CHUNKS,
    )(x, values)


def _idx_tc_body(nk, x_ref, vals_ref, idx_ref):
    xb = x_ref[...]
    vrow = vals_ref[...]
    v0 = vrow[0, 0]
    vlast = vrow[0, nk - 1]
    istep = jnp.float32(nk - 1) / (vlast - v0)
    t = (xb - v0) * istep
    kf = jnp.floor(t)
    ki = jnp.clip(kf.astype(jnp.int32), 0, nk - 2)
    frac = t - ki.astype(jnp.float32)
    idx_ref[...] = ki + (frac > 0.5).astype(jnp.int32)


@functools.partial(jax.jit, static_argnums=(0, 1))
def _idx_tc_call(n, nk, x, vals_row):
    rows = n // _TC_COLS
    grid = rows // _TC_BLOCK_ROWS
    x2 = x.reshape(rows, _TC_COLS)
    out = pl.pallas_call(
        functools.partial(_idx_tc_body, nk),
        out_shape=jax.ShapeDtypeStruct((rows, _TC_COLS), jnp.int32),
        grid=(grid,),
        in_specs=[
            pl.BlockSpec((_TC_BLOCK_ROWS, _TC_COLS), lambda i: (i, 0)),
            pl.BlockSpec((8, 128), lambda i: (0, 0)),
        ],
        out_specs=pl.BlockSpec((_TC_BLOCK_ROWS, _TC_COLS), lambda i: (i, 0)),
    )(x2, vals_row)
    return out.reshape(n)


def kernel(x, values):
    n = x.shape[0]
    nk = values.shape[0]
    vals_row = jnp.zeros((8, 128), jnp.float32).at[0, :nk].set(values)
    idx = _idx_tc_call(n, nk, x, vals_row)
    zc, q, qh = _vq_call(n, nk, x, values)
    return (zc, q, qh, idx)


# speedup vs baseline: 1.1159x; 1.1159x over previous
import functools

import jax
import jax.numpy as jnp
from jax import lax
from jax.experimental import pallas as pl
from jax.experimental.pallas import tpu as pltpu
from jax.experimental.pallas import tpu_sc as plsc

_NC = 2
_NS = 16
_L = 16
_NW = _NC * _NS
_CHUNKS = 4

_TC_COLS = 256
_TC_BLOCK_ROWS = 1024


def _vq_body(nk, per_w, x_hbm, vals_hbm,
             zc_hbm, q_hbm, qh_hbm, x_v, q_v, vals_v,
             sem_out, *sems_in):
    wid = lax.axis_index("c") * _NS + lax.axis_index("s")
    base = wid * per_w
    ch = per_w // _CHUNKS

    in_copies = []
    for c in range(_CHUNKS):
        in_copies.append(pltpu.async_copy(
            x_hbm.at[pl.ds(base + c * ch, ch)],
            x_v.at[pl.ds(c * ch, ch)],
            sems_in[c]))
    pltpu.sync_copy(vals_hbm, vals_v)

    vmin = jnp.full((_L,), jnp.inf, jnp.float32)
    vmax = jnp.full((_L,), -jnp.inf, jnp.float32)
    for j in range(nk // _L):
        vj = vals_v[pl.ds(j * _L, _L)]
        vmin = jnp.minimum(vmin, vj)
        vmax = jnp.maximum(vmax, vj)
    v0s = jnp.min(vmin)
    v63s = jnp.max(vmax)
    zf = jnp.zeros((_L,), jnp.float32)
    v0 = zf + v0s
    istep = (zf + jnp.float32(nk - 1)) / (zf + (v63s - v0s))

    out_copies = []
    for c in range(_CHUNKS):
        in_copies[c].wait()
        sl_v = pl.ds(c * ch, ch)
        sl_h = pl.ds(base + c * ch, ch)
        out_copies.append(pltpu.async_copy(x_v.at[sl_v], zc_hbm.at[sl_h], sem_out))

        @plsc.parallel_loop(c * (ch // _L), (c + 1) * (ch // _L), unroll=4)
        def body(i):
            s = pl.ds(i * _L, _L)
            xv = x_v[s]
            t = (xv - v0) * istep
            ki = jnp.clip(t.astype(jnp.int32), 0, nk - 2)
            k1 = ki + 1
            vk = plsc.load_gather(vals_v, [ki])
            vk1 = plsc.load_gather(vals_v, [k1])
            m = jnp.abs(xv - vk) <= jnp.abs(xv - vk1)
            q_v[s] = jnp.where(m, vk, vk1)

        out_copies.append(pltpu.async_copy(q_v.at[sl_v], q_hbm.at[sl_h], sem_out))
        out_copies.append(pltpu.async_copy(q_v.at[sl_v], qh_hbm.at[sl_h], sem_out))
    for cp in out_copies:
        cp.wait()


@functools.partial(jax.jit, static_argnums=(0, 1))
def _vq_call(n, nk, x, values):
    per_w = n // _NW
    mesh = plsc.VectorSubcoreMesh(core_axis_name="c", subcore_axis_name="s")
    return pl.kernel(
        functools.partial(_vq_body, nk, per_w),
        out_type=(
            jax.ShapeDtypeStruct((n,), jnp.float32),
            jax.ShapeDtypeStruct((n,), jnp.float32),
            jax.ShapeDtypeStruct((n,), jnp.float32),
        ),
        mesh=mesh,
        compiler_params=pltpu.CompilerParams(needs_layout_passes=False),
        scratch_types=[
            pltpu.VMEM((per_w,), jnp.float32),
            pltpu.VMEM((per_w,), jnp.float32),
            pltpu.VMEM((nk,), jnp.float32),
            pltpu.SemaphoreType.DMA,
        ] + [pltpu.SemaphoreType.DMA] * _CHUNKS,
    )(x, values)


def _idx_tc_body(nk, x_ref, vals_ref, idx_ref):
    xb = x_ref[...]
    vrow = vals_ref[...]
    v0 = vrow[0]
    vlast = vrow[nk - 1]
    istep = jnp.float32(nk - 1) / (vlast - v0)
    t = (xb - v0) * istep
    kf = jnp.floor(t)
    ki = jnp.clip(kf.astype(jnp.int32), 0, nk - 2)
    frac = t - ki.astype(jnp.float32)
    idx_ref[...] = ki + (frac > 0.5).astype(jnp.int32)


_TC_BLK = 131072


@functools.partial(jax.jit, static_argnums=(0, 1))
def _idx_tc_call(n, nk, x, values):
    grid = n // _TC_BLK
    return pl.pallas_call(
        functools.partial(_idx_tc_body, nk),
        out_shape=jax.ShapeDtypeStruct((n,), jnp.int32),
        grid=(grid,),
        in_specs=[
            pl.BlockSpec((_TC_BLK,), lambda i: (i,)),
            pl.BlockSpec((nk,), lambda i: (0,)),
        ],
        out_specs=pl.BlockSpec((_TC_BLK,), lambda i: (i,)),
    )(x, values)


def kernel(x, values):
    n = x.shape[0]
    nk = values.shape[0]
    idx = _idx_tc_call(n, nk, x, values)
    zc, q, qh = _vq_call(n, nk, x, values)
    return (zc, q, qh, idx)
